# baseline (speedup 1.0000x reference)
import functools

import jax
import jax.numpy as jnp
from jax import lax
from jax.experimental import pallas as pl
from jax.experimental.pallas import tpu as pltpu
from jax.experimental.pallas import tpu_sc as plsc

N_NODES = 10000
N_REL = 8
N_BASES = 4
D = 256
L = 16
NPAD = 10240
NC = 2
NS = 16
NW = NC * NS
CH = 128
GRP = 8
ACC_ROWS = N_REL * NPAD
ROWS_PER_TILE = ACC_ROWS // NS
ZROWS = 256
BN = 256
NPASS = D // L
RING = 16
LAG = 8


def _mesh():
    return plsc.VectorSubcoreMesh(
        core_axis_name="c", subcore_axis_name="s", num_cores=NC, num_subcores=NS)


@functools.cache
def _make_counts_kernel(nch0, nch1):
    @functools.partial(
        pl.kernel,
        out_type=jax.ShapeDtypeStruct((NC, ACC_ROWS, L), jnp.float32),
        mesh=_mesh(),
        scratch_types=[
            pltpu.VMEM((max(nch0, nch1), CH), jnp.int32),
            pltpu.VMEM((CH, L), jnp.float32),
            pltpu.VMEM((ZROWS, L), jnp.float32),
            pltpu.VMEM_SHARED((ACC_ROWS, L), jnp.float32),
            pltpu.SemaphoreType.DMA,
        ],
        compiler_params=pltpu.CompilerParams(use_tc_tiling_on_sc=False),
    )
    def cntk(sidx_hbm, cnt_out, sidx_v, ones_v, zbuf, acc, sem):
        cid = lax.axis_index("c")
        sid = lax.axis_index("s")
        wid = sid * NC + cid
        nch_w = jnp.where(cid == 0, nch0, nch1)
        pltpu.sync_copy(sidx_hbm.at[wid], sidx_v)

        def setz(i, _):
            zbuf[i, :] = jnp.zeros((L,), jnp.float32)
            return 0
        lax.fori_loop(0, ZROWS, setz, 0)

        def set1(i, _):
            ones_v[i, :] = jnp.ones((L,), jnp.float32)
            return 0
        lax.fori_loop(0, CH, set1, 0)

        base = sid * ROWS_PER_TILE

        def clr(i, _):
            pltpu.sync_copy(zbuf, acc.at[pl.ds(base + i * ZROWS, ZROWS)])
            return 0
        lax.fori_loop(0, ROWS_PER_TILE // ZROWS, clr, 0)
        plsc.subcore_barrier()

        def cfire(j, _):
            pltpu.async_copy(ones_v, acc.at[sidx_v.at[j]], sem, add=True)
            return 0
        lax.fori_loop(0, nch_w, cfire, 0)

        def cdrain(j, _):
            pltpu.make_async_copy(ones_v, acc.at[sidx_v.at[j]], sem).wait()
            return 0
        lax.fori_loop(0, nch_w, cdrain, 0)
        plsc.subcore_barrier()
        pltpu.sync_copy(acc.at[pl.ds(base, ROWS_PER_TILE)],
                        cnt_out.at[cid, pl.ds(base, ROWS_PER_TILE)])
    return cntk


@functools.cache
def _make_agg_kernel(nch0, nch1):
    nch = max(nch0, nch1)
    @functools.partial(
        pl.kernel,
        out_type=jax.ShapeDtypeStruct((NC, NPASS, ACC_ROWS, L), jnp.float32),
        mesh=_mesh(),
        scratch_types=[
            pltpu.VMEM((nch, CH), jnp.int32),
            pltpu.VMEM((nch, CH), jnp.int32),
            pltpu.VMEM((RING, CH, L), jnp.float32),
            pltpu.VMEM((ZROWS, L), jnp.float32),
            pltpu.VMEM_SHARED((ACC_ROWS, L), jnp.float32),
            pltpu.SemaphoreType.DMA,
            pltpu.SemaphoreType.DMA,
        ],
        compiler_params=pltpu.CompilerParams(use_tc_tiling_on_sc=False),
    )
    def aggk(xflat, src16_hbm, sidx_hbm, agg_out,
             idx_v, sidx_v, rows, zbuf, acc, sem_g, sem_s):
        cid = lax.axis_index("c")
        sid = lax.axis_index("s")
        wid = sid * NC + cid
        nch_w = jnp.where(cid == 0, nch0, nch1)
        pltpu.sync_copy(src16_hbm.at[wid], idx_v)
        pltpu.sync_copy(sidx_hbm.at[wid], sidx_v)

        def setz(i, _):
            zbuf[i, :] = jnp.zeros((L,), jnp.float32)
            return 0
        lax.fori_loop(0, ZROWS, setz, 0)

        base = sid * ROWS_PER_TILE
        nclr = ROWS_PER_TILE // ZROWS

        def gfire(j):
            pltpu.async_copy(
                xflat.at[idx_v.at[j]], rows.at[lax.rem(j, RING)], sem_g)

        def gwait(j):
            pltpu.make_async_copy(
                xflat.at[idx_v.at[j]], rows.at[lax.rem(j, RING)], sem_g).wait()

        def sfire(j):
            pltpu.async_copy(
                rows.at[lax.rem(j, RING)], acc.at[sidx_v.at[j]], sem_s,
                add=True)

        def swait(j):
            pltpu.make_async_copy(
                rows.at[lax.rem(j, RING)], acc.at[sidx_v.at[j]], sem_s).wait()

        def pass_body(dp, _):
            def clr_f(i, _):
                pltpu.async_copy(
                    zbuf, acc.at[pl.ds(base + i * ZROWS, ZROWS)], sem_g)
                return 0
            lax.fori_loop(0, nclr, clr_f, 0)

            def clr_d(i, _):
                pltpu.make_async_copy(
                    zbuf, acc.at[pl.ds(base + i * ZROWS, ZROWS)], sem_g).wait()
                return 0
            lax.fori_loop(0, nclr, clr_d, 0)
            plsc.subcore_barrier()

            def p1(j, _):
                gfire(j)
                return 0
            lax.fori_loop(0, LAG, p1, 0)

            def p2(j, _):
                gwait(j)
                sfire(j)
                gfire(j + LAG)
                return 0
            lax.fori_loop(0, RING - LAG, p2, 0)

            def p3(j, _):
                gwait(j)
                sfire(j)
                swait(j - LAG)
                gfire(j + LAG)
                return 0
            lax.fori_loop(RING - LAG, nch_w - LAG, p3, 0)

            def p4(j, _):
                gwait(j)
                sfire(j)
                return 0
            lax.fori_loop(nch_w - LAG, nch_w, p4, 0)

            def inc(j, _):
                for c in range(CH // L):
                    sl = pl.ds(c * L, L)
                    idx_v[j, sl] = idx_v[j, sl] + 1
                return 0
            lax.fori_loop(0, nch_w, inc, 0)

            def p5(j, _):
                swait(j)
                return 0
            lax.fori_loop(nch_w - RING, nch_w, p5, 0)
            plsc.subcore_barrier()
            pltpu.sync_copy(acc.at[pl.ds(base, ROWS_PER_TILE)],
                            agg_out.at[cid, dp, pl.ds(base, ROWS_PER_TILE)])
            return 0
        lax.fori_loop(0, NPASS, pass_body, 0)
    return aggk


def _dense_body(comp_ref, x_ref, agg_ref, cnt_ref, root_ref, basis_ref,
                bias_ref, gamma_ref, beta_ref, out_ref):
    x = x_ref[...]
    acc = jnp.dot(x, root_ref[...], preferred_element_type=jnp.float32)
    acc = acc + bias_ref[...]
    cnt = cnt_ref[0] + cnt_ref[1]
    rec = 1.0 / jnp.clip(cnt[..., :1], 1.0, None)
    agg = agg_ref[0] + agg_ref[1]
    aggm = [agg[r] * rec[r] for r in range(N_REL)]
    for b in range(N_BASES):
        cb = aggm[0] * comp_ref[0, b]
        for r in range(1, N_REL):
            cb = cb + aggm[r] * comp_ref[r, b]
        acc = acc + jnp.dot(cb, basis_ref[b], preferred_element_type=jnp.float32)
    mu = jnp.mean(acc, axis=-1, keepdims=True)
    xc = acc - mu
    var = jnp.mean(xc * xc, axis=-1, keepdims=True)
    y = xc * lax.rsqrt(var + 1e-5) * gamma_ref[...] + beta_ref[...]
    out_ref[...] = jnp.where(y >= 0, y, 0.1 * y)


@functools.cache
def _make_dense():
    return pl.pallas_call(
        _dense_body,
        grid=(NPAD // BN,),
        in_specs=[
            pl.BlockSpec(memory_space=pltpu.SMEM),
            pl.BlockSpec((BN, D), lambda i: (i, 0)),
            pl.BlockSpec((NC, N_REL, BN, D), lambda i: (0, 0, i, 0)),
            pl.BlockSpec((NC, N_REL, BN, L), lambda i: (0, 0, i, 0)),
            pl.BlockSpec((D, D), lambda i: (0, 0)),
            pl.BlockSpec((N_BASES, D, D), lambda i: (0, 0, 0)),
            pl.BlockSpec((1, D), lambda i: (0, 0)),
            pl.BlockSpec((1, D), lambda i: (0, 0)),
            pl.BlockSpec((1, D), lambda i: (0, 0)),
        ],
        out_specs=pl.BlockSpec((BN, D), lambda i: (i, 0)),
        out_shape=jax.ShapeDtypeStruct((NPAD, D), jnp.float32),
    )


def _split_edges(flat, nch0, nch1):
    nch = max(nch0, nch1)
    e0 = NS * nch0 * CH
    p0 = flat[:e0].reshape(NS, nch0, CH)
    p1 = flat[e0:].reshape(NS, nch1, CH)
    p0 = jnp.pad(p0, ((0, 0), (0, nch - nch0), (0, 0)))
    p1 = jnp.pad(p1, ((0, 0), (0, nch - nch1), (0, 0)))
    out = jnp.zeros((NW, nch, CH), flat.dtype)
    return out.at[0::2].set(p0).at[1::2].set(p1)


def kernel(entity, train_pos_edge_index, train_pos_edge_types,
           basis0, comp0, root0, bias0, gamma0, beta0,
           basis1, comp1, root1, bias1, gamma1, beta1):
    E = train_pos_edge_types.shape[0]
    src = train_pos_edge_index[0].astype(jnp.int32)
    dst = train_pos_edge_index[1].astype(jnp.int32)
    et = train_pos_edge_types.astype(jnp.int32)

    pairs = -(-E // (NS * CH))
    epad = pairs * NS * CH
    nch0 = max(RING, round(pairs * 0.39))
    nch1 = pairs - nch0
    pad = epad - E
    src16_flat = jnp.concatenate([src * L, jnp.zeros((pad,), jnp.int32)])
    sidx_flat = jnp.concatenate(
        [et * NPAD + dst, jnp.full((pad,), N_NODES, jnp.int32)])
    src16 = _split_edges(src16_flat, nch0, nch1)
    sidx = _split_edges(sidx_flat, nch0, nch1)

    xp = jnp.pad(entity, ((0, NPAD - N_NODES), (0, 0)))
    cnt = _make_counts_kernel(nch0, nch1)(sidx).reshape(NC, N_REL, NPAD, L)
    aggk = _make_agg_kernel(nch0, nch1)
    dense = _make_dense()
    for (basis, comp, root, bias, gamma, beta) in (
            (basis0, comp0, root0, bias0, gamma0, beta0),
            (basis1, comp1, root1, bias1, gamma1, beta1)):
        agg = aggk(xp.reshape(NPAD * L, L), src16, sidx)
        agg = agg.reshape(NC, NPASS, N_REL, NPAD, L).transpose(
            0, 2, 3, 1, 4).reshape(NC, N_REL, NPAD, D)
        xp = dense(comp, xp, agg, cnt, root, basis,
                   bias.reshape(1, D), gamma.reshape(1, D), beta.reshape(1, D))
    return xp[:N_NODES]


# speedup vs baseline: 1.5886x; 1.5886x over previous
import functools

import jax
import jax.numpy as jnp
from jax import lax
from jax.experimental import pallas as pl
from jax.experimental.pallas import tpu as pltpu
from jax.experimental.pallas import tpu_sc as plsc

N_NODES = 10000
N_REL = 8
N_BASES = 4
D = 256
L = 16
NPAD = 10240
NC = 2
NS = 16
NW = NC * NS
CH = 128
GRP = 8
ACC_ROWS = N_REL * NPAD
ROWS_PER_TILE = ACC_ROWS // NS
ZROWS = 256
BN = 256
NPASS = D // L
RING = 16
LAG = 8


def _mesh():
    return plsc.VectorSubcoreMesh(
        core_axis_name="c", subcore_axis_name="s", num_cores=NC, num_subcores=NS)


@functools.cache
def _make_counts_kernel(nch0, nch1):
    @functools.partial(
        pl.kernel,
        out_type=jax.ShapeDtypeStruct((NC, ACC_ROWS, L), jnp.float32),
        mesh=_mesh(),
        scratch_types=[
            pltpu.VMEM((max(nch0, nch1), CH), jnp.int32),
            pltpu.VMEM((CH, L), jnp.float32),
            pltpu.VMEM((ZROWS, L), jnp.float32),
            pltpu.VMEM_SHARED((ACC_ROWS, L), jnp.float32),
            pltpu.SemaphoreType.DMA,
        ],
        compiler_params=pltpu.CompilerParams(use_tc_tiling_on_sc=False),
    )
    def cntk(sidx_hbm, cnt_out, sidx_v, ones_v, zbuf, acc, sem):
        cid = lax.axis_index("c")
        sid = lax.axis_index("s")
        wid = sid * NC + cid
        nch_w = jnp.where(cid == 0, nch0, nch1)
        pltpu.sync_copy(sidx_hbm.at[wid], sidx_v)

        def setz(i, _):
            zbuf[i, :] = jnp.zeros((L,), jnp.float32)
            return 0
        lax.fori_loop(0, ZROWS, setz, 0)

        def set1(i, _):
            ones_v[i, :] = jnp.ones((L,), jnp.float32)
            return 0
        lax.fori_loop(0, CH, set1, 0)

        base = sid * ROWS_PER_TILE

        def clr(i, _):
            pltpu.sync_copy(zbuf, acc.at[pl.ds(base + i * ZROWS, ZROWS)])
            return 0
        lax.fori_loop(0, ROWS_PER_TILE // ZROWS, clr, 0)
        plsc.subcore_barrier()

        def cfire(j, _):
            pltpu.async_copy(ones_v, acc.at[sidx_v.at[j]], sem, add=True)
            return 0
        lax.fori_loop(0, nch_w, cfire, 0)

        def cdrain(j, _):
            pltpu.make_async_copy(ones_v, acc.at[sidx_v.at[j]], sem).wait()
            return 0
        lax.fori_loop(0, nch_w, cdrain, 0)
        plsc.subcore_barrier()
        pltpu.sync_copy(acc.at[pl.ds(base, ROWS_PER_TILE)],
                        cnt_out.at[cid, pl.ds(base, ROWS_PER_TILE)])
    return cntk


@functools.cache
def _make_agg_kernel(nch0, nch1):
    nch = max(nch0, nch1)
    @functools.partial(
        pl.kernel,
        out_type=jax.ShapeDtypeStruct((NC, ACC_ROWS, L, L), jnp.float32),
        mesh=_mesh(),
        scratch_types=[
            pltpu.VMEM((nch, CH), jnp.int32),
            pltpu.VMEM((nch, CH), jnp.int32),
            pltpu.VMEM((RING, CH, L), jnp.float32),
            pltpu.VMEM((ZROWS, L), jnp.float32),
            pltpu.VMEM_SHARED((ACC_ROWS, L), jnp.float32),
            pltpu.SemaphoreType.DMA,
            pltpu.SemaphoreType.DMA,
            pltpu.SemaphoreType.DMA,
        ],
        compiler_params=pltpu.CompilerParams(use_tc_tiling_on_sc=False),
    )
    def aggk(xflat, src16_hbm, sidx_hbm, agg_out,
             idx_v, sidx_v, rows, zbuf, acc, sem_g, sem_s, sem_d):
        cid = lax.axis_index("c")
        sid = lax.axis_index("s")
        wid = sid * NC + cid
        nch_w = jnp.where(cid == 0, nch0, nch1)
        pltpu.sync_copy(src16_hbm.at[wid], idx_v)
        pltpu.sync_copy(sidx_hbm.at[wid], sidx_v)

        def setz(i, _):
            zbuf[i, :] = jnp.zeros((L,), jnp.float32)
            return 0
        lax.fori_loop(0, ZROWS, setz, 0)

        base = sid * ROWS_PER_TILE
        nclr = ROWS_PER_TILE // ZROWS

        def gfire(j):
            pltpu.async_copy(
                xflat.at[idx_v.at[j]], rows.at[lax.rem(j, RING)], sem_g)

        def gwait(j):
            pltpu.make_async_copy(
                xflat.at[idx_v.at[j]], rows.at[lax.rem(j, RING)], sem_g).wait()

        def sfire(j):
            pltpu.async_copy(
                rows.at[lax.rem(j, RING)], acc.at[sidx_v.at[j]], sem_s,
                add=True)

        def swait(j):
            pltpu.make_async_copy(
                rows.at[lax.rem(j, RING)], acc.at[sidx_v.at[j]], sem_s).wait()

        dseg = ROWS_PER_TILE // 4

        def dump_fire(dp):
            for k in range(4):
                pltpu.async_copy(
                    acc.at[pl.ds(base + k * dseg, dseg)],
                    agg_out.at[cid, pl.ds(base + k * dseg, dseg), dp], sem_d)

        def dump_wait(dp):
            for k in range(4):
                pltpu.make_async_copy(
                    acc.at[pl.ds(base + k * dseg, dseg)],
                    agg_out.at[cid, pl.ds(base + k * dseg, dseg), dp],
                    sem_d).wait()

        def pass_body(dp, _):
            def p1(j, _):
                gfire(j)
                return 0
            lax.fori_loop(0, LAG, p1, 0)

            @pl.when(dp > 0)
            def _():
                dump_wait(dp - 1)

            def clr_f(i, _):
                pltpu.async_copy(
                    zbuf, acc.at[pl.ds(base + i * ZROWS, ZROWS)], sem_s)
                return 0
            lax.fori_loop(0, nclr, clr_f, 0)

            def clr_d(i, _):
                pltpu.make_async_copy(
                    zbuf, acc.at[pl.ds(base + i * ZROWS, ZROWS)], sem_s).wait()
                return 0
            lax.fori_loop(0, nclr, clr_d, 0)
            plsc.subcore_barrier()

            def p2(j, _):
                gwait(j)
                sfire(j)
                gfire(j + LAG)
                return 0
            lax.fori_loop(0, RING - LAG, p2, 0)

            def p3(j, _):
                gwait(j)
                sfire(j)
                swait(j - LAG)
                gfire(j + LAG)
                return 0
            lax.fori_loop(RING - LAG, nch_w - LAG, p3, 0)

            def p4(j, _):
                gwait(j)
                sfire(j)
                return 0
            lax.fori_loop(nch_w - LAG, nch_w, p4, 0)

            def inc(j, _):
                for c in range(CH // L):
                    sl = pl.ds(c * L, L)
                    idx_v[j, sl] = idx_v[j, sl] + 1
                return 0
            lax.fori_loop(0, nch_w, inc, 0)

            def p5(j, _):
                swait(j)
                return 0
            lax.fori_loop(nch_w - RING, nch_w, p5, 0)
            plsc.subcore_barrier()
            dump_fire(dp)
            return 0
        lax.fori_loop(0, NPASS, pass_body, 0)
        dump_wait(NPASS - 1)
    return aggk


def _dense_body(comp_ref, x_ref, agg_ref, cnt_ref, root_ref, basis_ref,
                bias_ref, gamma_ref, beta_ref, out_ref):
    x = x_ref[...]
    acc = jnp.dot(x, root_ref[...], preferred_element_type=jnp.float32)
    acc = acc + bias_ref[...]
    cnt = cnt_ref[0] + cnt_ref[1]
    rec = 1.0 / jnp.clip(cnt[..., :1], 1.0, None)
    agg = agg_ref[0] + agg_ref[1]
    aggm = [agg[r] * rec[r] for r in range(N_REL)]
    for b in range(N_BASES):
        cb = aggm[0] * comp_ref[0, b]
        for r in range(1, N_REL):
            cb = cb + aggm[r] * comp_ref[r, b]
        acc = acc + jnp.dot(cb, basis_ref[b], preferred_element_type=jnp.float32)
    mu = jnp.mean(acc, axis=-1, keepdims=True)
    xc = acc - mu
    var = jnp.mean(xc * xc, axis=-1, keepdims=True)
    y = xc * lax.rsqrt(var + 1e-5) * gamma_ref[...] + beta_ref[...]
    out_ref[...] = jnp.where(y >= 0, y, 0.1 * y)


@functools.cache
def _make_dense():
    return pl.pallas_call(
        _dense_body,
        grid=(NPAD // BN,),
        in_specs=[
            pl.BlockSpec(memory_space=pltpu.SMEM),
            pl.BlockSpec((BN, D), lambda i: (i, 0)),
            pl.BlockSpec((NC, N_REL, BN, D), lambda i: (0, 0, i, 0)),
            pl.BlockSpec((NC, N_REL, BN, L), lambda i: (0, 0, i, 0)),
            pl.BlockSpec((D, D), lambda i: (0, 0)),
            pl.BlockSpec((N_BASES, D, D), lambda i: (0, 0, 0)),
            pl.BlockSpec((1, D), lambda i: (0, 0)),
            pl.BlockSpec((1, D), lambda i: (0, 0)),
            pl.BlockSpec((1, D), lambda i: (0, 0)),
        ],
        out_specs=pl.BlockSpec((BN, D), lambda i: (i, 0)),
        out_shape=jax.ShapeDtypeStruct((NPAD, D), jnp.float32),
    )


def _split_edges(flat, nch0, nch1):
    nch = max(nch0, nch1)
    e0 = NS * nch0 * CH
    p0 = flat[:e0].reshape(NS, nch0, CH)
    p1 = flat[e0:].reshape(NS, nch1, CH)
    p0 = jnp.pad(p0, ((0, 0), (0, nch - nch0), (0, 0)))
    p1 = jnp.pad(p1, ((0, 0), (0, nch - nch1), (0, 0)))
    out = jnp.zeros((NW, nch, CH), flat.dtype)
    return out.at[0::2].set(p0).at[1::2].set(p1)


def kernel(entity, train_pos_edge_index, train_pos_edge_types,
           basis0, comp0, root0, bias0, gamma0, beta0,
           basis1, comp1, root1, bias1, gamma1, beta1):
    E = train_pos_edge_types.shape[0]
    src = train_pos_edge_index[0].astype(jnp.int32)
    dst = train_pos_edge_index[1].astype(jnp.int32)
    et = train_pos_edge_types.astype(jnp.int32)

    pairs = -(-E // (NS * CH))
    epad = pairs * NS * CH
    nch0 = max(RING, round(pairs * 0.39))
    nch1 = pairs - nch0
    pad = epad - E
    src16_flat = jnp.concatenate([src * L, jnp.zeros((pad,), jnp.int32)])
    sidx_flat = jnp.concatenate(
        [et * NPAD + dst, jnp.full((pad,), N_NODES, jnp.int32)])
    src16 = _split_edges(src16_flat, nch0, nch1)
    sidx = _split_edges(sidx_flat, nch0, nch1)

    xp = jnp.pad(entity, ((0, NPAD - N_NODES), (0, 0)))
    cnt = _make_counts_kernel(nch0, nch1)(sidx).reshape(NC, N_REL, NPAD, L)
    aggk = _make_agg_kernel(nch0, nch1)
    dense = _make_dense()
    for (basis, comp, root, bias, gamma, beta) in (
            (basis0, comp0, root0, bias0, gamma0, beta0),
            (basis1, comp1, root1, bias1, gamma1, beta1)):
        agg = aggk(xp.reshape(NPAD * L, L), src16, sidx)
        agg = agg.reshape(NC, N_REL, NPAD, D)
        xp = dense(comp, xp, agg, cnt, root, basis,
                   bias.reshape(1, D), gamma.reshape(1, D), beta.reshape(1, D))
    return xp[:N_NODES]


# speedup vs baseline: 1.6020x; 1.0085x over previous
import functools

import jax
import jax.numpy as jnp
from jax import lax
from jax.experimental import pallas as pl
from jax.experimental.pallas import tpu as pltpu
from jax.experimental.pallas import tpu_sc as plsc

N_NODES = 10000
N_REL = 8
N_BASES = 4
D = 256
L = 16
NPAD = 10240
NC = 2
NS = 16
NW = NC * NS
CH = 128
GRP = 8
ACC_ROWS = N_REL * NPAD
ROWS_PER_TILE = ACC_ROWS // NS
ZROWS = 128
BN = 256
NPASS = D // L
RING = 16
LAG = 8


def _mesh():
    return plsc.VectorSubcoreMesh(
        core_axis_name="c", subcore_axis_name="s", num_cores=NC, num_subcores=NS)


@functools.cache
def _make_counts_kernel(nch0, nch1):
    @functools.partial(
        pl.kernel,
        out_type=jax.ShapeDtypeStruct((NC, ACC_ROWS, L), jnp.float32),
        mesh=_mesh(),
        scratch_types=[
            pltpu.VMEM((max(nch0, nch1), CH), jnp.int32),
            pltpu.VMEM((CH, L), jnp.float32),
            pltpu.VMEM((ZROWS, L), jnp.float32),
            pltpu.VMEM_SHARED((ACC_ROWS, L), jnp.float32),
            pltpu.SemaphoreType.DMA,
        ],
        compiler_params=pltpu.CompilerParams(use_tc_tiling_on_sc=False),
    )
    def cntk(sidx_hbm, cnt_out, sidx_v, ones_v, zbuf, acc, sem):
        cid = lax.axis_index("c")
        sid = lax.axis_index("s")
        wid = sid * NC + cid
        nch_w = jnp.where(cid == 0, nch0, nch1)
        pltpu.sync_copy(sidx_hbm.at[wid], sidx_v)

        def setz(i, _):
            zbuf[i, :] = jnp.zeros((L,), jnp.float32)
            return 0
        lax.fori_loop(0, ZROWS, setz, 0)

        def set1(i, _):
            ones_v[i, :] = jnp.ones((L,), jnp.float32)
            return 0
        lax.fori_loop(0, CH, set1, 0)

        base = sid * ROWS_PER_TILE

        def clr(i, _):
            pltpu.sync_copy(zbuf, acc.at[pl.ds(base + i * ZROWS, ZROWS)])
            return 0
        lax.fori_loop(0, ROWS_PER_TILE // ZROWS, clr, 0)
        plsc.subcore_barrier()

        def cfire(j, _):
            pltpu.async_copy(ones_v, acc.at[sidx_v.at[j]], sem, add=True)
            return 0
        lax.fori_loop(0, nch_w, cfire, 0)

        def cdrain(j, _):
            pltpu.make_async_copy(ones_v, acc.at[sidx_v.at[j]], sem).wait()
            return 0
        lax.fori_loop(0, nch_w, cdrain, 0)
        plsc.subcore_barrier()
        pltpu.sync_copy(acc.at[pl.ds(base, ROWS_PER_TILE)],
                        cnt_out.at[cid, pl.ds(base, ROWS_PER_TILE)])
    return cntk


@functools.cache
def _make_agg_kernel(nch0, nch1):
    nch = max(nch0, nch1)
    @functools.partial(
        pl.kernel,
        out_type=jax.ShapeDtypeStruct((NC, ACC_ROWS, L, L), jnp.float32),
        mesh=_mesh(),
        scratch_types=[
            pltpu.VMEM((nch, CH), jnp.int32),
            pltpu.VMEM((nch, CH), jnp.int32),
            pltpu.VMEM((RING, CH, L), jnp.float32),
            pltpu.VMEM((ZROWS, L), jnp.float32),
            pltpu.VMEM_SHARED((ACC_ROWS, L), jnp.float32),
            pltpu.SemaphoreType.DMA,
            pltpu.SemaphoreType.DMA,
            pltpu.SemaphoreType.DMA,
        ],
        compiler_params=pltpu.CompilerParams(use_tc_tiling_on_sc=False),
    )
    def aggk(xflat, src16_hbm, sidx_hbm, agg_out,
             idx_v, sidx_v, rows, zbuf, acc, sem_g, sem_s, sem_d):
        cid = lax.axis_index("c")
        sid = lax.axis_index("s")
        wid = sid * NC + cid
        nch_w = jnp.where(cid == 0, nch0, nch1)
        pltpu.sync_copy(src16_hbm.at[wid], idx_v)
        pltpu.sync_copy(sidx_hbm.at[wid], sidx_v)

        def setz(i, _):
            zbuf[i, :] = jnp.zeros((L,), jnp.float32)
            return 0
        lax.fori_loop(0, ZROWS, setz, 0)

        base = sid * ROWS_PER_TILE
        nclr = ROWS_PER_TILE // ZROWS

        def gfire(j):
            pltpu.async_copy(
                xflat.at[idx_v.at[j]], rows.at[lax.rem(j, RING)], sem_g)

        def gwait(j):
            pltpu.make_async_copy(
                xflat.at[idx_v.at[j]], rows.at[lax.rem(j, RING)], sem_g).wait()

        def sfire(j):
            pltpu.async_copy(
                rows.at[lax.rem(j, RING)], acc.at[sidx_v.at[j]], sem_s,
                add=True)

        def swait(j):
            pltpu.make_async_copy(
                rows.at[lax.rem(j, RING)], acc.at[sidx_v.at[j]], sem_s).wait()

        dseg = ROWS_PER_TILE // 4

        def dump_fire(dp):
            for k in range(4):
                pltpu.async_copy(
                    acc.at[pl.ds(base + k * dseg, dseg)],
                    agg_out.at[cid, pl.ds(base + k * dseg, dseg), dp], sem_d)

        def dump_wait(dp):
            for k in range(4):
                pltpu.make_async_copy(
                    acc.at[pl.ds(base + k * dseg, dseg)],
                    agg_out.at[cid, pl.ds(base + k * dseg, dseg), dp],
                    sem_d).wait()

        def pass_body(dp, _):
            def p1(j, _):
                gfire(j)
                return 0
            lax.fori_loop(0, LAG, p1, 0)

            @pl.when(dp > 0)
            def _():
                dump_wait(dp - 1)

            def clr_f(i, _):
                pltpu.async_copy(
                    zbuf, acc.at[pl.ds(base + i * ZROWS, ZROWS)], sem_s)
                return 0
            lax.fori_loop(0, nclr, clr_f, 0)

            def clr_d(i, _):
                pltpu.make_async_copy(
                    zbuf, acc.at[pl.ds(base + i * ZROWS, ZROWS)], sem_s).wait()
                return 0
            lax.fori_loop(0, nclr, clr_d, 0)
            plsc.subcore_barrier()

            def p2(j, _):
                gwait(j)
                sfire(j)
                gfire(j + LAG)
                return 0
            lax.fori_loop(0, RING - LAG, p2, 0)

            def p3(j, _):
                gwait(j)
                sfire(j)
                swait(j - LAG)
                gfire(j + LAG)
                return 0
            lax.fori_loop(RING - LAG, nch_w - LAG, p3, 0)

            def p4(j, _):
                gwait(j)
                sfire(j)
                return 0
            lax.fori_loop(nch_w - LAG, nch_w, p4, 0)

            def inc(j, _):
                for c in range(CH // L):
                    sl = pl.ds(c * L, L)
                    idx_v[j, sl] = idx_v[j, sl] + 1
                return 0
            lax.fori_loop(0, nch_w, inc, 0)

            def p5(j, _):
                swait(j)
                return 0
            lax.fori_loop(nch_w - RING, nch_w, p5, 0)
            plsc.subcore_barrier()
            dump_fire(dp)
            return 0
        lax.fori_loop(0, NPASS, pass_body, 0)
        dump_wait(NPASS - 1)
    return aggk


def _dense_body(comp_ref, x_ref, agg_ref, cnt_ref, root_ref, basis_ref,
                bias_ref, gamma_ref, beta_ref, out_ref):
    x = x_ref[...]
    acc = jnp.dot(x, root_ref[...], preferred_element_type=jnp.float32)
    acc = acc + bias_ref[...]
    cnt = cnt_ref[0] + cnt_ref[1]
    rec = 1.0 / jnp.clip(cnt[..., :1], 1.0, None)
    agg = agg_ref[0] + agg_ref[1]
    aggm = [agg[r] * rec[r] for r in range(N_REL)]
    for b in range(N_BASES):
        cb = aggm[0] * comp_ref[0, b]
        for r in range(1, N_REL):
            cb = cb + aggm[r] * comp_ref[r, b]
        acc = acc + jnp.dot(cb, basis_ref[b], preferred_element_type=jnp.float32)
    mu = jnp.mean(acc, axis=-1, keepdims=True)
    xc = acc - mu
    var = jnp.mean(xc * xc, axis=-1, keepdims=True)
    y = xc * lax.rsqrt(var + 1e-5) * gamma_ref[...] + beta_ref[...]
    out_ref[...] = jnp.where(y >= 0, y, 0.1 * y)


@functools.cache
def _make_dense():
    return pl.pallas_call(
        _dense_body,
        grid=(NPAD // BN,),
        in_specs=[
            pl.BlockSpec(memory_space=pltpu.SMEM),
            pl.BlockSpec((BN, D), lambda i: (i, 0)),
            pl.BlockSpec((NC, N_REL, BN, D), lambda i: (0, 0, i, 0)),
            pl.BlockSpec((NC, N_REL, BN, L), lambda i: (0, 0, i, 0)),
            pl.BlockSpec((D, D), lambda i: (0, 0)),
            pl.BlockSpec((N_BASES, D, D), lambda i: (0, 0, 0)),
            pl.BlockSpec((1, D), lambda i: (0, 0)),
            pl.BlockSpec((1, D), lambda i: (0, 0)),
            pl.BlockSpec((1, D), lambda i: (0, 0)),
        ],
        out_specs=pl.BlockSpec((BN, D), lambda i: (i, 0)),
        out_shape=jax.ShapeDtypeStruct((NPAD, D), jnp.float32),
    )


def _split_edges(flat, nch0, nch1):
    nch = max(nch0, nch1)
    e0 = NS * nch0 * CH
    p0 = flat[:e0].reshape(NS, nch0, CH)
    p1 = flat[e0:].reshape(NS, nch1, CH)
    p0 = jnp.pad(p0, ((0, 0), (0, nch - nch0), (0, 0)))
    p1 = jnp.pad(p1, ((0, 0), (0, nch - nch1), (0, 0)))
    return jnp.stack([p0, p1], axis=1).reshape(NW, nch, CH)


def kernel(entity, train_pos_edge_index, train_pos_edge_types,
           basis0, comp0, root0, bias0, gamma0, beta0,
           basis1, comp1, root1, bias1, gamma1, beta1):
    E = train_pos_edge_types.shape[0]
    src = train_pos_edge_index[0].astype(jnp.int32)
    dst = train_pos_edge_index[1].astype(jnp.int32)
    et = train_pos_edge_types.astype(jnp.int32)

    pairs = -(-E // (NS * CH))
    epad = pairs * NS * CH
    nch0 = max(RING, round(pairs * 0.32))
    nch1 = pairs - nch0
    pad = epad - E
    src16_flat = jnp.concatenate([src * L, jnp.zeros((pad,), jnp.int32)])
    sidx_flat = jnp.concatenate(
        [et * NPAD + dst, jnp.full((pad,), N_NODES, jnp.int32)])
    src16 = _split_edges(src16_flat, nch0, nch1)
    sidx = _split_edges(sidx_flat, nch0, nch1)

    xp = jnp.pad(entity, ((0, NPAD - N_NODES), (0, 0)))
    cnt = _make_counts_kernel(nch0, nch1)(sidx).reshape(NC, N_REL, NPAD, L)
    aggk = _make_agg_kernel(nch0, nch1)
    dense = _make_dense()
    for (basis, comp, root, bias, gamma, beta) in (
            (basis0, comp0, root0, bias0, gamma0, beta0),
            (basis1, comp1, root1, bias1, gamma1, beta1)):
        agg = aggk(xp.reshape(NPAD * L, L), src16, sidx)
        agg = agg.reshape(NC, N_REL, NPAD, D)
        xp = dense(comp, xp, agg, cnt, root, basis,
                   bias.reshape(1, D), gamma.reshape(1, D), beta.reshape(1, D))
    return xp[:N_NODES]


# speedup vs baseline: 1.6104x; 1.0052x over previous
import functools

import jax
import jax.numpy as jnp
from jax import lax
from jax.experimental import pallas as pl
from jax.experimental.pallas import tpu as pltpu
from jax.experimental.pallas import tpu_sc as plsc

N_NODES = 10000
N_REL = 8
N_BASES = 4
D = 256
L = 16
NPAD = 10240
NC = 2
NS = 16
NW = NC * NS
CH = 128
GRP = 8
ACC_ROWS = N_REL * NPAD
ROWS_PER_TILE = ACC_ROWS // NS
ZROWS = 128
BN = 512
NPASS = D // L
RING = 16
LAG = 8


def _mesh():
    return plsc.VectorSubcoreMesh(
        core_axis_name="c", subcore_axis_name="s", num_cores=NC, num_subcores=NS)


@functools.cache
def _make_counts_kernel(nch0, nch1):
    @functools.partial(
        pl.kernel,
        out_type=jax.ShapeDtypeStruct((NC, ACC_ROWS, L), jnp.float32),
        mesh=_mesh(),
        scratch_types=[
            pltpu.VMEM((max(nch0, nch1), CH), jnp.int32),
            pltpu.VMEM((CH, L), jnp.float32),
            pltpu.VMEM((ZROWS, L), jnp.float32),
            pltpu.VMEM_SHARED((ACC_ROWS, L), jnp.float32),
            pltpu.SemaphoreType.DMA,
        ],
        compiler_params=pltpu.CompilerParams(use_tc_tiling_on_sc=False),
    )
    def cntk(sidx_hbm, cnt_out, sidx_v, ones_v, zbuf, acc, sem):
        cid = lax.axis_index("c")
        sid = lax.axis_index("s")
        wid = sid * NC + cid
        nch_w = jnp.where(cid == 0, nch0, nch1)
        pltpu.sync_copy(sidx_hbm.at[wid], sidx_v)

        def setz(i, _):
            zbuf[i, :] = jnp.zeros((L,), jnp.float32)
            return 0
        lax.fori_loop(0, ZROWS, setz, 0)

        def set1(i, _):
            ones_v[i, :] = jnp.ones((L,), jnp.float32)
            return 0
        lax.fori_loop(0, CH, set1, 0)

        base = sid * ROWS_PER_TILE

        def clr(i, _):
            pltpu.sync_copy(zbuf, acc.at[pl.ds(base + i * ZROWS, ZROWS)])
            return 0
        lax.fori_loop(0, ROWS_PER_TILE // ZROWS, clr, 0)
        plsc.subcore_barrier()

        def cfire(j, _):
            pltpu.async_copy(ones_v, acc.at[sidx_v.at[j]], sem, add=True)
            return 0
        lax.fori_loop(0, nch_w, cfire, 0)

        def cdrain(j, _):
            pltpu.make_async_copy(ones_v, acc.at[sidx_v.at[j]], sem).wait()
            return 0
        lax.fori_loop(0, nch_w, cdrain, 0)
        plsc.subcore_barrier()
        pltpu.sync_copy(acc.at[pl.ds(base, ROWS_PER_TILE)],
                        cnt_out.at[cid, pl.ds(base, ROWS_PER_TILE)])
    return cntk


@functools.cache
def _make_agg_kernel(nch0, nch1):
    nch = max(nch0, nch1)
    @functools.partial(
        pl.kernel,
        out_type=jax.ShapeDtypeStruct((NC, ACC_ROWS, L, L), jnp.float32),
        mesh=_mesh(),
        scratch_types=[
            pltpu.VMEM((nch, CH), jnp.int32),
            pltpu.VMEM((nch, CH), jnp.int32),
            pltpu.VMEM((RING, CH, L), jnp.float32),
            pltpu.VMEM((ZROWS, L), jnp.float32),
            pltpu.VMEM_SHARED((ACC_ROWS, L), jnp.float32),
            pltpu.SemaphoreType.DMA,
            pltpu.SemaphoreType.DMA,
            pltpu.SemaphoreType.DMA,
        ],
        compiler_params=pltpu.CompilerParams(use_tc_tiling_on_sc=False),
    )
    def aggk(xflat, src16_hbm, sidx_hbm, agg_out,
             idx_v, sidx_v, rows, zbuf, acc, sem_g, sem_s, sem_d):
        cid = lax.axis_index("c")
        sid = lax.axis_index("s")
        wid = sid * NC + cid
        nch_w = jnp.where(cid == 0, nch0, nch1)
        pltpu.sync_copy(src16_hbm.at[wid], idx_v)
        pltpu.sync_copy(sidx_hbm.at[wid], sidx_v)

        def setz(i, _):
            zbuf[i, :] = jnp.zeros((L,), jnp.float32)
            return 0
        lax.fori_loop(0, ZROWS, setz, 0)

        base = sid * ROWS_PER_TILE
        nclr = ROWS_PER_TILE // ZROWS

        def gfire(j):
            pltpu.async_copy(
                xflat.at[idx_v.at[j]], rows.at[lax.rem(j, RING)], sem_g)

        def gwait(j):
            pltpu.make_async_copy(
                xflat.at[idx_v.at[j]], rows.at[lax.rem(j, RING)], sem_g).wait()

        def sfire(j):
            pltpu.async_copy(
                rows.at[lax.rem(j, RING)], acc.at[sidx_v.at[j]], sem_s,
                add=True)

        def swait(j):
            pltpu.make_async_copy(
                rows.at[lax.rem(j, RING)], acc.at[sidx_v.at[j]], sem_s).wait()

        dseg = ROWS_PER_TILE // 4

        def dump_fire(dp):
            for k in range(4):
                pltpu.async_copy(
                    acc.at[pl.ds(base + k * dseg, dseg)],
                    agg_out.at[cid, pl.ds(base + k * dseg, dseg), dp], sem_d)

        def dump_wait(dp):
            for k in range(4):
                pltpu.make_async_copy(
                    acc.at[pl.ds(base + k * dseg, dseg)],
                    agg_out.at[cid, pl.ds(base + k * dseg, dseg), dp],
                    sem_d).wait()

        def pass_body(dp, _):
            def p1(j, _):
                gfire(j)
                return 0
            lax.fori_loop(0, LAG, p1, 0)

            @pl.when(dp > 0)
            def _():
                dump_wait(dp - 1)

            def clr_f(i, _):
                pltpu.async_copy(
                    zbuf, acc.at[pl.ds(base + i * ZROWS, ZROWS)], sem_s)
                return 0
            lax.fori_loop(0, nclr, clr_f, 0)

            def clr_d(i, _):
                pltpu.make_async_copy(
                    zbuf, acc.at[pl.ds(base + i * ZROWS, ZROWS)], sem_s).wait()
                return 0
            lax.fori_loop(0, nclr, clr_d, 0)
            plsc.subcore_barrier()

            def p2(j, _):
                gwait(j)
                sfire(j)
                gfire(j + LAG)
                return 0
            lax.fori_loop(0, RING - LAG, p2, 0)

            def p3(j, _):
                gwait(j)
                sfire(j)
                swait(j - LAG)
                gfire(j + LAG)
                return 0
            lax.fori_loop(RING - LAG, nch_w - LAG, p3, 0)

            def p4(j, _):
                gwait(j)
                sfire(j)
                return 0
            lax.fori_loop(nch_w - LAG, nch_w, p4, 0)

            def inc(j, _):
                for c in range(CH // L):
                    sl = pl.ds(c * L, L)
                    idx_v[j, sl] = idx_v[j, sl] + 1
                return 0
            lax.fori_loop(0, nch_w, inc, 0)

            def p5(j, _):
                swait(j)
                return 0
            lax.fori_loop(nch_w - RING, nch_w, p5, 0)
            plsc.subcore_barrier()
            dump_fire(dp)
            return 0
        lax.fori_loop(0, NPASS, pass_body, 0)
        dump_wait(NPASS - 1)
    return aggk


def _dense_body(comp_ref, x_ref, agg_ref, cnt_ref, root_ref, basis_ref,
                bias_ref, gamma_ref, beta_ref, out_ref):
    x = x_ref[...]
    acc = jnp.dot(x, root_ref[...], preferred_element_type=jnp.float32)
    acc = acc + bias_ref[...]
    cnt = cnt_ref[0] + cnt_ref[1]
    rec = 1.0 / jnp.clip(cnt[..., :1], 1.0, None)
    agg = agg_ref[0] + agg_ref[1]
    aggm = [agg[r] * rec[r] for r in range(N_REL)]
    for b in range(N_BASES):
        cb = aggm[0] * comp_ref[0, b]
        for r in range(1, N_REL):
            cb = cb + aggm[r] * comp_ref[r, b]
        acc = acc + jnp.dot(cb, basis_ref[b], preferred_element_type=jnp.float32)
    mu = jnp.mean(acc, axis=-1, keepdims=True)
    xc = acc - mu
    var = jnp.mean(xc * xc, axis=-1, keepdims=True)
    y = xc * lax.rsqrt(var + 1e-5) * gamma_ref[...] + beta_ref[...]
    out_ref[...] = jnp.where(y >= 0, y, 0.1 * y)


@functools.cache
def _make_dense():
    return pl.pallas_call(
        _dense_body,
        grid=(NPAD // BN,),
        in_specs=[
            pl.BlockSpec(memory_space=pltpu.SMEM),
            pl.BlockSpec((BN, D), lambda i: (i, 0)),
            pl.BlockSpec((NC, N_REL, BN, D), lambda i: (0, 0, i, 0)),
            pl.BlockSpec((NC, N_REL, BN, L), lambda i: (0, 0, i, 0)),
            pl.BlockSpec((D, D), lambda i: (0, 0)),
            pl.BlockSpec((N_BASES, D, D), lambda i: (0, 0, 0)),
            pl.BlockSpec((1, D), lambda i: (0, 0)),
            pl.BlockSpec((1, D), lambda i: (0, 0)),
            pl.BlockSpec((1, D), lambda i: (0, 0)),
        ],
        out_specs=pl.BlockSpec((BN, D), lambda i: (i, 0)),
        out_shape=jax.ShapeDtypeStruct((NPAD, D), jnp.float32),
    )


def _split_edges(flat, nch0, nch1):
    nch = max(nch0, nch1)
    e0 = NS * nch0 * CH
    p0 = flat[:e0].reshape(NS, nch0, CH)
    p1 = flat[e0:].reshape(NS, nch1, CH)
    p0 = jnp.pad(p0, ((0, 0), (0, nch - nch0), (0, 0)))
    p1 = jnp.pad(p1, ((0, 0), (0, nch - nch1), (0, 0)))
    return jnp.stack([p0, p1], axis=1).reshape(NW, nch, CH)


def kernel(entity, train_pos_edge_index, train_pos_edge_types,
           basis0, comp0, root0, bias0, gamma0, beta0,
           basis1, comp1, root1, bias1, gamma1, beta1):
    E = train_pos_edge_types.shape[0]
    src = train_pos_edge_index[0].astype(jnp.int32)
    dst = train_pos_edge_index[1].astype(jnp.int32)
    et = train_pos_edge_types.astype(jnp.int32)

    pairs = -(-E // (NS * CH))
    epad = pairs * NS * CH
    nch0 = max(RING, round(pairs * 0.32))
    nch1 = pairs - nch0
    pad = epad - E
    src16_flat = jnp.concatenate([src * L, jnp.zeros((pad,), jnp.int32)])
    sidx_flat = jnp.concatenate(
        [et * NPAD + dst, jnp.full((pad,), N_NODES, jnp.int32)])
    src16 = _split_edges(src16_flat, nch0, nch1)
    sidx = _split_edges(sidx_flat, nch0, nch1)

    xp = jnp.pad(entity, ((0, NPAD - N_NODES), (0, 0)))
    cnt = _make_counts_kernel(nch0, nch1)(sidx).reshape(NC, N_REL, NPAD, L)
    aggk = _make_agg_kernel(nch0, nch1)
    dense = _make_dense()
    for (basis, comp, root, bias, gamma, beta) in (
            (basis0, comp0, root0, bias0, gamma0, beta0),
            (basis1, comp1, root1, bias1, gamma1, beta1)):
        agg = aggk(xp.reshape(NPAD * L, L), src16, sidx)
        agg = agg.reshape(NC, N_REL, NPAD, D)
        xp = dense(comp, xp, agg, cnt, root, basis,
                   bias.reshape(1, D), gamma.reshape(1, D), beta.reshape(1, D))
    return xp[:N_NODES]
